# baseline (speedup 1.0000x reference)
import functools

import jax
import jax.numpy as jnp
from jax.experimental import pallas as pl
from jax.experimental.pallas import tpu as pltpu


def _fused_kernel(x_ref, sel_ref, st_ref, ew_ref, eb_ref, g1_ref, be1_ref,
                  b1_ref, g2_ref, be2_ref, b2_ref, w1_ref, w2_ref, out_ref,
                  *, C, Sp, num_groups, eps):
    f32 = jnp.float32
    P = Sp * Sp * Sp
    plane = Sp * Sp
    line = Sp
    hw = 4 * plane
    R = C * Sp
    hi = jax.lax.Precision.HIGHEST

    xa = x_ref[0]

    rs = jnp.sum(xa, axis=1, keepdims=True)
    rq = jnp.sum(xa * xa, axis=1, keepdims=True)
    gr = R // num_groups
    gi = jax.lax.broadcasted_iota(jnp.int32, (num_groups, R), 0)
    gj = jax.lax.broadcasted_iota(jnp.int32, (num_groups, R), 1) // gr
    gmap = jnp.where(gi == gj, 1.0, 0.0).astype(f32)
    ti = jax.lax.broadcasted_iota(jnp.int32, (R, num_groups), 0) // gr
    tj = jax.lax.broadcasted_iota(jnp.int32, (R, num_groups), 1)
    gmap_t = jnp.where(ti == tj, 1.0, 0.0).astype(f32)
    n1 = float(gr * 2 * hw)
    gs = jnp.dot(gmap, rs, precision=hi, preferred_element_type=f32) / n1
    gq = jnp.dot(gmap, rq, precision=hi, preferred_element_type=f32) / n1
    ginv = jax.lax.rsqrt(gq - gs * gs + eps)
    mu_r = jnp.dot(gmap_t, gs, precision=hi, preferred_element_type=f32)
    inv_r = jnp.dot(gmap_t, ginv, precision=hi, preferred_element_type=f32)
    ga = inv_r * g1_ref[...]
    bb = be1_ref[...] - mu_r * ga

    y = xa * ga + bb
    y = y * jax.nn.sigmoid(y)
    q = y[:, :hw] + y[:, hw:]
    qs = xa[:, :hw] + xa[:, hw:]
    big = jnp.concatenate([q, qs], axis=0)

    si = jax.lax.broadcasted_iota(jnp.int32, (hw, plane), 0)
    sj = jax.lax.broadcasted_iota(jnp.int32, (hw, plane), 1)
    smat = jnp.where(((si // (4 * line)) == (sj // line))
                     & (((si % (2 * line)) // 2) == (sj % line)),
                     0.125, 0.0).astype(f32)
    p2 = jnp.dot(big, smat, preferred_element_type=f32)

    p2pad = jnp.concatenate(
        [p2, jnp.zeros((Sp, plane), f32)], axis=0)
    sel0 = sel_ref[...]
    parts = []
    for dp in range(Sp):
        parts.append(jnp.dot(sel0, p2pad[dp:dp + 2 * R, :],
                             preferred_element_type=f32))
    pall = jnp.concatenate(parts, axis=1)
    p = pall[:C, :]
    ps = pall[C:, :]

    st = st_ref[0]
    st = st * jax.nn.sigmoid(st)
    emb = jnp.dot(ew_ref[...], st, precision=hi,
                  preferred_element_type=f32) + eb_ref[...]
    scale = emb[:C, :]
    shift = emb[C:, :]
    a_f = g2_ref[...] * (1.0 + scale)
    b_f = be2_ref[...] * (1.0 + scale) + shift

    padw = ((2 * (plane + line + 1) + 127) // 128) * 128

    def conv27(src, w_ref):
        full = jnp.concatenate(
            [src, jnp.zeros((C, padw), dtype=src.dtype)], axis=1)
        rows = []
        for kd in range(3):
            for kh in range(3):
                for kw in range(3):
                    off = kd * plane + kh * line + kw
                    rows.append(full[:, off:off + P])
        a = jnp.concatenate(rows, axis=0)
        return jnp.dot(w_ref[...], a, preferred_element_type=f32)

    y1 = conv27(p.astype(jnp.bfloat16), w1_ref) + b1_ref[...]
    v1 = Sp - 2
    cgc = C // num_groups
    ii = jax.lax.broadcasted_iota(jnp.int32, (1, P), 1)
    valid = ((ii // plane < v1) & ((ii // line) % Sp < v1)
             & (ii % Sp < v1))
    mf = jnp.where(valid, 1.0, 0.0).astype(f32)
    ri = jax.lax.broadcasted_iota(jnp.int32, (C, C), 0) // cgc
    ci = jax.lax.broadcasted_iota(jnp.int32, (C, C), 1) // cgc
    cmix = jnp.where(ri == ci, 1.0, 0.0).astype(f32)
    ym = y1 * mf
    s2 = jnp.sum(ym, axis=1, keepdims=True)
    q2 = jnp.sum(ym * y1, axis=1, keepdims=True)
    n2 = float(cgc * v1 * v1 * v1)
    mu2 = jnp.dot(cmix, s2, precision=hi, preferred_element_type=f32) / n2
    ex22 = jnp.dot(cmix, q2, precision=hi, preferred_element_type=f32) / n2
    inv2 = jax.lax.rsqrt(ex22 - mu2 * mu2 + eps)
    za = inv2 * a_f
    zb = b_f - mu2 * za
    z = y1 * za + zb
    z = z * jax.nn.sigmoid(z)

    y2 = conv27(z.astype(jnp.bfloat16), w2_ref)
    pspad = jnp.concatenate([ps, jnp.zeros((C, padw), f32)], axis=1)
    soff = 2 * plane + 2 * line + 2
    out = y2 + b2_ref[...] + pspad[:, soff:soff + P]
    out_ref[0] = out[:, :(Sp - 4) * plane].astype(out_ref.dtype)


def kernel(x, style, embed_w, embed_b, gn1_gamma, gn1_beta, conv1_w, conv1_b,
           gn2_gamma, gn2_beta, conv2_w, conv2_b):
    num_groups = 16
    eps = 1e-6
    B, C, D, H, W = x.shape
    E = style.shape[1]
    Sp = D // 2
    P = Sp * Sp * Sp
    R = C * Sp
    f32 = jnp.float32

    x2 = x.reshape(B, R, 2 * H * W)

    rr = jnp.arange(2 * C).reshape(2 * C, 1)
    cc = jnp.arange(2 * R).reshape(1, 2 * R)
    sel0 = (cc == (rr % C) * Sp + (rr // C) * R).astype(f32)

    w1m = jnp.transpose(conv1_w, (0, 2, 3, 4, 1)).reshape(C, 27 * C)
    w2m = jnp.transpose(conv2_w, (0, 2, 3, 4, 1)).reshape(C, 27 * C)
    w1m = w1m.astype(jnp.bfloat16)
    w2m = w2m.astype(jnp.bfloat16)

    st_t = style.reshape(B, E, 1).astype(f32)
    ebc = embed_b.reshape(2 * C, 1).astype(f32)
    g1r = jnp.repeat(gn1_gamma, Sp).reshape(R, 1).astype(f32)
    be1r = jnp.repeat(gn1_beta, Sp).reshape(R, 1).astype(f32)
    b1c = conv1_b.reshape(C, 1).astype(f32)
    g2c = gn2_gamma.reshape(C, 1).astype(f32)
    be2c = gn2_beta.reshape(C, 1).astype(f32)
    b2c = conv2_b.reshape(C, 1).astype(f32)

    def bcast(shape):
        return pl.BlockSpec(shape, lambda b: tuple(0 for _ in shape))

    out = pl.pallas_call(
        functools.partial(_fused_kernel, C=C, Sp=Sp,
                          num_groups=num_groups, eps=eps),
        grid=(B,),
        in_specs=[
            pl.BlockSpec((1, R, 2 * H * W), lambda b: (b, 0, 0)),
            bcast((2 * C, 2 * R)),
            pl.BlockSpec((1, E, 1), lambda b: (b, 0, 0)),
            bcast((2 * C, E)),
            bcast((2 * C, 1)),
            bcast((R, 1)),
            bcast((R, 1)),
            bcast((C, 1)),
            bcast((C, 1)),
            bcast((C, 1)),
            bcast((C, 1)),
            bcast((C, 27 * C)),
            bcast((C, 27 * C)),
        ],
        out_specs=pl.BlockSpec((1, C, (Sp - 4) * Sp * Sp),
                               lambda b: (b, 0, 0)),
        out_shape=jax.ShapeDtypeStruct((B, C, (Sp - 4) * Sp * Sp), f32),
        compiler_params=pltpu.CompilerParams(
            dimension_semantics=("parallel",),
            vmem_limit_bytes=128 * 1024 * 1024,
        ),
    )(x2, sel0, st_t, embed_w.astype(f32), ebc, g1r, be1r, b1c, g2c, be2c,
      b2c, w1m, w2m)

    v2 = Sp - 4
    return out.reshape(B, C, v2, Sp, Sp)[:, :, :, :v2, :v2]


# speedup vs baseline: 1.5975x; 1.5975x over previous
import functools

import jax
import jax.numpy as jnp
from jax.experimental import pallas as pl
from jax.experimental.pallas import tpu as pltpu


def _fused_kernel(x_ref, st_ref, ew_ref, eb_ref, g1_ref, be1_ref, b1_ref,
                  g2_ref, be2_ref, b2_ref, w1_ref, w2_ref, out_ref,
                  *, C, Sp, num_groups, eps):
    f32 = jnp.float32
    P = Sp * Sp * Sp
    plane = Sp * Sp
    line = Sp
    cg = C // num_groups
    hi = jax.lax.Precision.HIGHEST

    xs = [x_ref[0, i] for i in range(8)]
    xsum = xs[0]
    for xi in xs[1:]:
        xsum = xsum + xi
    xsq = xs[0] * xs[0]
    for xi in xs[1:]:
        xsq = xsq + xi * xi

    sc = jnp.sum(xsum, axis=1, keepdims=True)
    sq = jnp.sum(xsq, axis=1, keepdims=True)
    ri = jax.lax.broadcasted_iota(jnp.int32, (C, C), 0) // cg
    ci = jax.lax.broadcasted_iota(jnp.int32, (C, C), 1) // cg
    gmat = jnp.where(ri == ci, 1.0, 0.0).astype(f32)
    n1 = float(cg * 8 * P)
    mu = jnp.dot(gmat, sc, precision=hi, preferred_element_type=f32) / n1
    ex2 = jnp.dot(gmat, sq, precision=hi, preferred_element_type=f32) / n1
    inv = jax.lax.rsqrt(ex2 - mu * mu + eps)
    ga = inv * g1_ref[...]
    bb = be1_ref[...] - mu * ga

    p = None
    for xi in xs:
        t = xi * ga + bb
        t = t * jax.nn.sigmoid(t)
        p = t if p is None else p + t
    p = p * 0.125
    ps = xsum * 0.125

    st = st_ref[0]
    st = st * jax.nn.sigmoid(st)
    emb = jnp.dot(ew_ref[...], st, precision=hi,
                  preferred_element_type=f32) + eb_ref[...]
    scale = emb[:C, :]
    shift = emb[C:, :]
    a_f = g2_ref[...] * (1.0 + scale)
    b_f = be2_ref[...] * (1.0 + scale) + shift

    padw = ((2 * (plane + line + 1) + 127) // 128) * 128

    def conv27(src, w_ref):
        full = jnp.concatenate(
            [src, jnp.zeros((C, padw), dtype=src.dtype)], axis=1)
        nchunk = 4
        cw = P // nchunk
        outs = []
        for ci_ in range(nchunk):
            rows = []
            for kd in range(3):
                for kh in range(3):
                    for kw in range(3):
                        off = ci_ * cw + kd * plane + kh * line + kw
                        rows.append(full[:, off:off + cw])
            a = jnp.concatenate(rows, axis=0)
            outs.append(jnp.dot(w_ref[...], a, preferred_element_type=f32))
        return jnp.concatenate(outs, axis=1)

    y1 = conv27(p.astype(jnp.bfloat16), w1_ref) + b1_ref[...]
    v1 = Sp - 2
    ii = jax.lax.broadcasted_iota(jnp.int32, (1, P), 1)
    valid = ((ii // plane < v1) & ((ii // line) % Sp < v1)
             & (ii % Sp < v1))
    mf = jnp.where(valid, 1.0, 0.0).astype(f32)
    ym = y1 * mf
    s2 = jnp.sum(ym, axis=1, keepdims=True)
    q2 = jnp.sum(ym * y1, axis=1, keepdims=True)
    n2 = float(cg * v1 * v1 * v1)
    mu2 = jnp.dot(gmat, s2, precision=hi, preferred_element_type=f32) / n2
    ex22 = jnp.dot(gmat, q2, precision=hi, preferred_element_type=f32) / n2
    inv2 = jax.lax.rsqrt(ex22 - mu2 * mu2 + eps)
    za = inv2 * a_f
    zb = b_f - mu2 * za
    z = y1 * za + zb
    z = z * jax.nn.sigmoid(z)

    y2 = conv27(z.astype(jnp.bfloat16), w2_ref)
    pspad = jnp.concatenate([ps, jnp.zeros((C, padw), f32)], axis=1)
    soff = 2 * plane + 2 * line + 2
    out = y2 + b2_ref[...] + pspad[:, soff:soff + P]
    out_ref[0] = out[:, :(Sp - 4) * plane].astype(out_ref.dtype)


def kernel(x, style, embed_w, embed_b, gn1_gamma, gn1_beta, conv1_w, conv1_b,
           gn2_gamma, gn2_beta, conv2_w, conv2_b):
    num_groups = 16
    eps = 1e-6
    B, C, D, H, W = x.shape
    E = style.shape[1]
    Sp = D // 2
    P = Sp * Sp * Sp
    f32 = jnp.float32

    xc = x.reshape(B, C, Sp, 2, Sp, 2, Sp, 2)
    xc = jnp.transpose(xc, (0, 3, 5, 7, 1, 2, 4, 6))
    xc = xc.reshape(B, 8, C, P)

    w1m = jnp.transpose(conv1_w, (0, 2, 3, 4, 1)).reshape(C, 27 * C)
    w2m = jnp.transpose(conv2_w, (0, 2, 3, 4, 1)).reshape(C, 27 * C)
    w1m = w1m.astype(jnp.bfloat16)
    w2m = w2m.astype(jnp.bfloat16)

    st_t = style.reshape(B, E, 1).astype(f32)
    ebc = embed_b.reshape(2 * C, 1).astype(f32)
    g1c = gn1_gamma.reshape(C, 1).astype(f32)
    be1c = gn1_beta.reshape(C, 1).astype(f32)
    b1c = conv1_b.reshape(C, 1).astype(f32)
    g2c = gn2_gamma.reshape(C, 1).astype(f32)
    be2c = gn2_beta.reshape(C, 1).astype(f32)
    b2c = conv2_b.reshape(C, 1).astype(f32)

    def bcast(shape):
        return pl.BlockSpec(shape, lambda b: tuple(0 for _ in shape))

    out = pl.pallas_call(
        functools.partial(_fused_kernel, C=C, Sp=Sp,
                          num_groups=num_groups, eps=eps),
        grid=(B,),
        in_specs=[
            pl.BlockSpec((1, 8, C, P), lambda b: (b, 0, 0, 0)),
            pl.BlockSpec((1, E, 1), lambda b: (b, 0, 0)),
            bcast((2 * C, E)),
            bcast((2 * C, 1)),
            bcast((C, 1)),
            bcast((C, 1)),
            bcast((C, 1)),
            bcast((C, 1)),
            bcast((C, 1)),
            bcast((C, 1)),
            bcast((C, 27 * C)),
            bcast((C, 27 * C)),
        ],
        out_specs=pl.BlockSpec((1, C, (Sp - 4) * Sp * Sp),
                               lambda b: (b, 0, 0)),
        out_shape=jax.ShapeDtypeStruct((B, C, (Sp - 4) * Sp * Sp), f32),
        compiler_params=pltpu.CompilerParams(
            dimension_semantics=("parallel",),
            vmem_limit_bytes=128 * 1024 * 1024,
        ),
    )(xc, st_t, embed_w.astype(f32), ebc, g1c, be1c, b1c, g2c, be2c, b2c,
      w1m, w2m)

    v2 = Sp - 4
    return out.reshape(B, C, v2, Sp, Sp)[:, :, :, :v2, :v2]
